# baseline (speedup 1.0000x reference)
import functools

import jax
import jax.numpy as jnp
from jax import lax
from jax.experimental import pallas as pl
from jax.experimental.pallas import tpu as pltpu
from jax.experimental.pallas import tpu_sc as plsc

EMBED = 64
LANES = 16
NCORES = 2
NSUB = 16
NWORKERS = NCORES * NSUB
WAVE = 32


def _body(bpw, tid_hbm, cid_hbm, table_in, table_out, o_hbm,
          ids_t, ids_c, buf_t, buf_c, tpose, out_v, sem0, sem1, sem2, sem3):
    sems = (sem0, sem1, sem2, sem3)
    wid = lax.axis_index("s") * NCORES + lax.axis_index("c")
    base = wid * bpw

    pltpu.sync_copy(tid_hbm.at[pl.ds(base, bpw)], ids_t)
    pltpu.sync_copy(cid_hbm.at[pl.ds(base, bpw)], ids_c)

    iota = lax.iota(jnp.int32, LANES)

    def wave_body(w, carry):
        wbase = w * WAVE
        copies = []
        for g in range(WAVE // LANES):
            idt16 = ids_t[pl.ds(wbase + g * LANES, LANES)]
            idc16 = ids_c[pl.ds(wbase + g * LANES, LANES)]
            for r in range(LANES):
                i = g * LANES + r
                copies.append(pltpu.async_copy(
                    table_in.at[idt16[r]], buf_t.at[i], sems[(2 * i) % 4]))
                copies.append(pltpu.async_copy(
                    table_out.at[idc16[r]], buf_c.at[i],
                    sems[(2 * i + 1) % 4]))
        for cp in copies:
            cp.wait()
        for g in range(WAVE // LANES):
            for r in range(LANES):
                i = g * LANES + r
                acc = buf_t[i, pl.ds(0, LANES)] * buf_c[i, pl.ds(0, LANES)]
                for c in range(1, EMBED // LANES):
                    acc = acc + (buf_t[i, pl.ds(c * LANES, LANES)] *
                                 buf_c[i, pl.ds(c * LANES, LANES)])
                plsc.store_scatter(tpose, [iota * LANES + r], acc)
            colsum = tpose[pl.ds(0, LANES)]
            for l in range(1, LANES):
                colsum = colsum + tpose[pl.ds(l * LANES, LANES)]
            out_v[pl.ds(wbase + g * LANES, LANES)] = colsum
        return carry

    lax.fori_loop(0, bpw // WAVE, wave_body, 0)
    pltpu.sync_copy(out_v, o_hbm.at[pl.ds(base, bpw)])


def kernel(target_ids, context_ids, in_embed, out_embed):
    batch = target_ids.shape[0]
    bpw = batch // NWORKERS
    mesh = plsc.VectorSubcoreMesh(core_axis_name="c", subcore_axis_name="s")
    f = pl.kernel(
        functools.partial(_body, bpw),
        out_type=jax.ShapeDtypeStruct((batch,), jnp.float32),
        mesh=mesh,
        scratch_types=[
            pltpu.VMEM((bpw,), jnp.int32),
            pltpu.VMEM((bpw,), jnp.int32),
            pltpu.VMEM((WAVE, EMBED), jnp.float32),
            pltpu.VMEM((WAVE, EMBED), jnp.float32),
            pltpu.VMEM((LANES * LANES,), jnp.float32),
            pltpu.VMEM((bpw,), jnp.float32),
            pltpu.SemaphoreType.DMA,
            pltpu.SemaphoreType.DMA,
            pltpu.SemaphoreType.DMA,
            pltpu.SemaphoreType.DMA,
        ],
        compiler_params=pltpu.CompilerParams(needs_layout_passes=False),
    )
    return f(target_ids.astype(jnp.int32), context_ids.astype(jnp.int32),
             in_embed, out_embed)
